# baseline (device time: 421801 ns/iter reference)
import jax
import jax.numpy as jnp
from jax import lax
from jax.experimental import pallas as pl
from jax.experimental.pallas import tpu as pltpu

N_DEV = 32


def kernel(x, w_mat):
    m_global, k_per = x.shape
    _, n = w_mat.shape
    m_per = m_global // N_DEV

    def body(x_ref, w_ref, out_ref, comm_ref, send_sems, recv_sems):
        my = lax.axis_index("i")
        left = lax.rem(my - 1 + N_DEV, N_DEV)
        right = lax.rem(my + 1, N_DEV)

        barrier_sem = pltpu.get_barrier_semaphore()
        for nbr in (left, right):
            pl.semaphore_signal(
                barrier_sem, inc=1,
                device_id=(nbr,), device_id_type=pl.DeviceIdType.MESH,
            )
        pl.semaphore_wait(barrier_sem, 2)

        def partial(c):
            xb = x_ref[pl.ds(c * m_per, m_per), :]
            return jnp.dot(xb, w_ref[:, :], preferred_element_type=jnp.float32)

        c0 = lax.rem(my - 1 + N_DEV, N_DEV)
        comm_ref[0, :, :] = partial(c0)

        for h in range(N_DEV - 1):
            send_slot = h % 2
            recv_slot = (h + 1) % 2
            rdma = pltpu.make_async_remote_copy(
                src_ref=comm_ref.at[send_slot],
                dst_ref=comm_ref.at[recv_slot],
                send_sem=send_sems.at[send_slot],
                recv_sem=recv_sems.at[recv_slot],
                device_id=(right,),
                device_id_type=pl.DeviceIdType.MESH,
            )
            rdma.start()
            rdma.wait()

            c = lax.rem(my - 2 - h + 2 * N_DEV, N_DEV)
            if h < N_DEV - 2:
                comm_ref[recv_slot, :, :] = comm_ref[recv_slot, :, :] + partial(c)
            else:
                out_ref[:, :] = comm_ref[recv_slot, :, :] + partial(c)

    return pl.pallas_call(
        body,
        out_shape=jax.ShapeDtypeStruct((m_per, n), jnp.float32),
        in_specs=[
            pl.BlockSpec(memory_space=pltpu.VMEM),
            pl.BlockSpec(memory_space=pltpu.VMEM),
        ],
        out_specs=pl.BlockSpec(memory_space=pltpu.VMEM),
        scratch_shapes=[
            pltpu.VMEM((2, m_per, n), jnp.float32),
            pltpu.SemaphoreType.DMA((2,)),
            pltpu.SemaphoreType.DMA((2,)),
        ],
        compiler_params=pltpu.CompilerParams(collective_id=0),
    )(x, w_mat)


# device time: 229008 ns/iter; 1.8419x vs baseline; 1.8419x over previous
import jax
import jax.numpy as jnp
from jax import lax
from jax.experimental import pallas as pl
from jax.experimental.pallas import tpu as pltpu

N_DEV = 32


def kernel(x, w_mat):
    m_global, k_per = x.shape
    _, n = w_mat.shape
    m_per = m_global // N_DEV
    half = n // 2

    def body(x_ref, w_ref, out_ref, comm_r, comm_l,
             send_r, recv_r, send_l, recv_l):
        my = lax.axis_index("i")
        left = lax.rem(my - 1 + N_DEV, N_DEV)
        right = lax.rem(my + 1, N_DEV)

        barrier_sem = pltpu.get_barrier_semaphore()
        for nbr in (left, right):
            pl.semaphore_signal(
                barrier_sem, inc=1,
                device_id=(nbr,), device_id_type=pl.DeviceIdType.MESH,
            )
        pl.semaphore_wait(barrier_sem, 2)

        def partial(c, col0):
            xb = x_ref[pl.ds(c * m_per, m_per), :]
            return jnp.dot(
                xb, w_ref[:, col0:col0 + half],
                preferred_element_type=jnp.float32,
            )

        c_r0 = lax.rem(my - 1 + N_DEV, N_DEV)
        c_l0 = lax.rem(my + 1, N_DEV)
        comm_r[0, :, :] = partial(c_r0, 0).astype(jnp.bfloat16)
        comm_l[0, :, :] = partial(c_l0, half).astype(jnp.bfloat16)

        for h in range(N_DEV - 1):
            ss = h % 2
            rs = (h + 1) % 2
            rdma_r = pltpu.make_async_remote_copy(
                src_ref=comm_r.at[ss],
                dst_ref=comm_r.at[rs],
                send_sem=send_r.at[ss],
                recv_sem=recv_r.at[rs],
                device_id=(right,),
                device_id_type=pl.DeviceIdType.MESH,
            )
            rdma_l = pltpu.make_async_remote_copy(
                src_ref=comm_l.at[ss],
                dst_ref=comm_l.at[rs],
                send_sem=send_l.at[ss],
                recv_sem=recv_l.at[rs],
                device_id=(left,),
                device_id_type=pl.DeviceIdType.MESH,
            )
            rdma_r.start()
            rdma_l.start()

            c_r = lax.rem(my - 2 - h + 2 * N_DEV, N_DEV)
            c_l = lax.rem(my + 2 + h, N_DEV)
            p_r = partial(c_r, 0)
            p_l = partial(c_l, half)

            rdma_r.wait()
            rdma_l.wait()
            if h < N_DEV - 2:
                comm_r[rs, :, :] = (
                    comm_r[rs, :, :].astype(jnp.float32) + p_r
                ).astype(jnp.bfloat16)
                comm_l[rs, :, :] = (
                    comm_l[rs, :, :].astype(jnp.float32) + p_l
                ).astype(jnp.bfloat16)
            else:
                out_ref[:, :half] = comm_r[rs, :, :].astype(jnp.float32) + p_r
                out_ref[:, half:] = comm_l[rs, :, :].astype(jnp.float32) + p_l

    return pl.pallas_call(
        body,
        out_shape=jax.ShapeDtypeStruct((m_per, n), jnp.float32),
        in_specs=[
            pl.BlockSpec(memory_space=pltpu.VMEM),
            pl.BlockSpec(memory_space=pltpu.VMEM),
        ],
        out_specs=pl.BlockSpec(memory_space=pltpu.VMEM),
        scratch_shapes=[
            pltpu.VMEM((2, m_per, half), jnp.bfloat16),
            pltpu.VMEM((2, m_per, half), jnp.bfloat16),
            pltpu.SemaphoreType.DMA((2,)),
            pltpu.SemaphoreType.DMA((2,)),
            pltpu.SemaphoreType.DMA((2,)),
            pltpu.SemaphoreType.DMA((2,)),
        ],
        compiler_params=pltpu.CompilerParams(collective_id=0),
    )(x, w_mat)


# device time: 155324 ns/iter; 2.7156x vs baseline; 1.4744x over previous
import jax
import jax.numpy as jnp
from jax import lax
from jax.experimental import pallas as pl
from jax.experimental.pallas import tpu as pltpu

N_DEV = 32


def _snake_coords():
    coords = []
    for z in range(4):
        for y in range(4):
            xs = (0, 1) if y % 2 == 0 else (1, 0)
            for x in xs:
                coords.append((x, y, z))
    return coords


_C16 = [(0, 0), (0, 1), (0, 2), (0, 3), (1, 3), (1, 2), (1, 1), (2, 1),
        (2, 2), (2, 3), (3, 3), (3, 2), (3, 1), (3, 0), (2, 0), (1, 0)]


def _ham_ring():
    ring = []
    for k, (y, z) in enumerate(_C16):
        if k % 2 == 0:
            ring += [(0, y, z), (1, y, z)]
        else:
            ring += [(1, y, z), (0, y, z)]
    return ring


_RING = _ham_ring()
for _a, _b in zip(_RING, _RING[1:] + _RING[:1]):
    assert sum(abs(p - q) for p, q in zip(_a, _b)) == 1, (_a, _b)

_POS_OF = {c: p for p, c in enumerate(_snake_coords())}
RING_TO_POS = [_POS_OF[c] for c in _RING]
POS_TO_RING = [0] * N_DEV
for _r, _p in enumerate(RING_TO_POS):
    POS_TO_RING[_p] = _r

def kernel(x, w_mat):
    m_global, k_per = x.shape
    _, n = w_mat.shape
    m_per = m_global // N_DEV
    half = n // 2

    r2p = jnp.asarray(RING_TO_POS, dtype=jnp.int32)
    p2r = jnp.asarray(POS_TO_RING, dtype=jnp.int32)

    def body(r2p_ref, p2r_ref, x_ref, w_ref, out_ref, comm_r, comm_l,
             send_r, recv_r, send_l, recv_l):
        my = lax.axis_index("i")
        k = p2r_ref[my]

        def ring_pos(delta):
            return r2p_ref[lax.rem(k + delta + 2 * N_DEV, N_DEV)]

        right = ring_pos(1)
        left = ring_pos(-1)

        barrier_sem = pltpu.get_barrier_semaphore()
        for nbr in (left, right):
            pl.semaphore_signal(
                barrier_sem, inc=1,
                device_id=(nbr,), device_id_type=pl.DeviceIdType.MESH,
            )
        pl.semaphore_wait(barrier_sem, 2)

        def partial(c, col0):
            xb = x_ref[pl.ds(c * m_per, m_per), :]
            return jnp.dot(
                xb, w_ref[:, col0:col0 + half],
                preferred_element_type=jnp.float32,
            )

        comm_r[0, :, :] = partial(ring_pos(-1), 0).astype(jnp.bfloat16)
        comm_l[0, :, :] = partial(ring_pos(1), half).astype(jnp.bfloat16)

        for h in range(N_DEV - 1):
            ss = h % 2
            rs = (h + 1) % 2
            rdma_r = pltpu.make_async_remote_copy(
                src_ref=comm_r.at[ss],
                dst_ref=comm_r.at[rs],
                send_sem=send_r.at[ss],
                recv_sem=recv_r.at[rs],
                device_id=(right,),
                device_id_type=pl.DeviceIdType.MESH,
            )
            rdma_l = pltpu.make_async_remote_copy(
                src_ref=comm_l.at[ss],
                dst_ref=comm_l.at[rs],
                send_sem=send_l.at[ss],
                recv_sem=recv_l.at[rs],
                device_id=(left,),
                device_id_type=pl.DeviceIdType.MESH,
            )
            rdma_r.start()
            rdma_l.start()

            p_r = partial(ring_pos(-2 - h), 0)
            p_l = partial(ring_pos(2 + h), half)

            rdma_r.wait()
            rdma_l.wait()
            if h < N_DEV - 2:
                comm_r[rs, :, :] = (
                    comm_r[rs, :, :].astype(jnp.float32) + p_r
                ).astype(jnp.bfloat16)
                comm_l[rs, :, :] = (
                    comm_l[rs, :, :].astype(jnp.float32) + p_l
                ).astype(jnp.bfloat16)
            else:
                out_ref[:, :half] = comm_r[rs, :, :].astype(jnp.float32) + p_r
                out_ref[:, half:] = comm_l[rs, :, :].astype(jnp.float32) + p_l

    return pl.pallas_call(
        body,
        out_shape=jax.ShapeDtypeStruct((m_per, n), jnp.float32),
        in_specs=[
            pl.BlockSpec(memory_space=pltpu.SMEM),
            pl.BlockSpec(memory_space=pltpu.SMEM),
            pl.BlockSpec(memory_space=pltpu.VMEM),
            pl.BlockSpec(memory_space=pltpu.VMEM),
        ],
        out_specs=pl.BlockSpec(memory_space=pltpu.VMEM),
        scratch_shapes=[
            pltpu.VMEM((2, m_per, half), jnp.bfloat16),
            pltpu.VMEM((2, m_per, half), jnp.bfloat16),
            pltpu.SemaphoreType.DMA((2,)),
            pltpu.SemaphoreType.DMA((2,)),
            pltpu.SemaphoreType.DMA((2,)),
            pltpu.SemaphoreType.DMA((2,)),
        ],
        compiler_params=pltpu.CompilerParams(collective_id=0),
    )(r2p, p2r, x, w_mat)


# device time: 99573 ns/iter; 4.2361x vs baseline; 1.5599x over previous
import jax
import jax.numpy as jnp
from jax import lax
from jax.experimental import pallas as pl
from jax.experimental.pallas import tpu as pltpu

N_DEV = 32
NSEG = 4
NBUF = 4


def _snake_coords():
    coords = []
    for z in range(4):
        for y in range(4):
            xs = (0, 1) if y % 2 == 0 else (1, 0)
            for x in xs:
                coords.append((x, y, z))
    return coords


_C16 = [(0, 0), (0, 1), (0, 2), (0, 3), (1, 3), (1, 2), (1, 1), (2, 1),
        (2, 2), (2, 3), (3, 3), (3, 2), (3, 1), (3, 0), (2, 0), (1, 0)]


def _ham_ring():
    ring = []
    for k, (y, z) in enumerate(_C16):
        if k % 2 == 0:
            ring += [(0, y, z), (1, y, z)]
        else:
            ring += [(1, y, z), (0, y, z)]
    return ring


_RING = _ham_ring()
for _a, _b in zip(_RING, _RING[1:] + _RING[:1]):
    assert sum(abs(p - q) for p, q in zip(_a, _b)) == 1, (_a, _b)

_POS_OF = {c: p for p, c in enumerate(_snake_coords())}
RING_TO_POS = [_POS_OF[c] for c in _RING]
POS_TO_RING = [0] * N_DEV
for _r, _p in enumerate(RING_TO_POS):
    POS_TO_RING[_p] = _r


def kernel(x, w_mat):
    m_global, k_per = x.shape
    _, n = w_mat.shape
    m_per = m_global // N_DEV
    half = n // 2
    segw = half // NSEG

    r2p = jnp.asarray(RING_TO_POS, dtype=jnp.int32)
    p2r = jnp.asarray(POS_TO_RING, dtype=jnp.int32)

    def body(r2p_ref, p2r_ref, x_ref, w_ref, out_ref, buf_r, buf_l,
             send_r, recv_r, send_l, recv_l, cred_r, cred_l):
        my = lax.axis_index("i")
        k = p2r_ref[my]

        def ring_pos(delta):
            return r2p_ref[lax.rem(k + delta + 2 * N_DEV, N_DEV)]

        right = ring_pos(1)
        left = ring_pos(-1)

        barrier_sem = pltpu.get_barrier_semaphore()
        for nbr in (left, right):
            pl.semaphore_signal(
                barrier_sem, inc=1,
                device_id=(nbr,), device_id_type=pl.DeviceIdType.MESH,
            )
        pl.semaphore_wait(barrier_sem, 2)

        def partial(c, col0):
            xb = x_ref[pl.ds(c * m_per, m_per), :]
            return jnp.dot(
                xb, w_ref[:, col0:col0 + half],
                preferred_element_type=jnp.float32,
            )

        def mk(buf, ssem, rsem, j4, s, dev):
            return pltpu.make_async_remote_copy(
                src_ref=buf.at[j4, :, s * segw:(s + 1) * segw],
                dst_ref=buf.at[(j4 + 1) % NBUF, :, s * segw:(s + 1) * segw],
                send_sem=ssem.at[s, j4 % 2],
                recv_sem=rsem.at[s, j4 % 2],
                device_id=(dev,),
                device_id_type=pl.DeviceIdType.MESH,
            )

        def hop(h, h4, mid, is_last, credit=True):
            p_r = partial(ring_pos(-2 - h), 0)
            p_l = partial(ring_pos(2 + h), half)
            d4 = (h4 + 1) % NBUF

            for s in range(NSEG):
                cols = slice(s * segw, (s + 1) * segw)

                mk(buf_r, send_r, recv_r, h4, s, right).wait_recv()
                if mid:
                    mk(buf_r, send_r, recv_r, (h4 + 3) % NBUF, s,
                       right).wait_send()
                    pl.semaphore_wait(cred_r.at[s, (h4 + 1) % 2], 1)
                if not is_last:
                    buf_r[d4, :, cols] = (
                        buf_r[d4, :, cols].astype(jnp.float32)
                        + p_r[:, cols]
                    ).astype(jnp.bfloat16)
                    mk(buf_r, send_r, recv_r, d4, s, right).start()
                else:
                    out_ref[:, cols] = (
                        buf_r[d4, :, cols].astype(jnp.float32)
                        + p_r[:, cols]
                    )
                if credit:
                    pl.semaphore_signal(
                        cred_r.at[s, h4 % 2], inc=1,
                        device_id=(left,),
                        device_id_type=pl.DeviceIdType.MESH,
                    )

                mk(buf_l, send_l, recv_l, h4, s, left).wait_recv()
                if mid:
                    mk(buf_l, send_l, recv_l, (h4 + 3) % NBUF, s,
                       left).wait_send()
                    pl.semaphore_wait(cred_l.at[s, (h4 + 1) % 2], 1)
                if not is_last:
                    buf_l[d4, :, cols] = (
                        buf_l[d4, :, cols].astype(jnp.float32)
                        + p_l[:, cols]
                    ).astype(jnp.bfloat16)
                    mk(buf_l, send_l, recv_l, d4, s, left).start()
                else:
                    out_ref[:, half + s * segw:half + (s + 1) * segw] = (
                        buf_l[d4, :, cols].astype(jnp.float32)
                        + p_l[:, cols]
                    )
                if credit:
                    pl.semaphore_signal(
                        cred_l.at[s, h4 % 2], inc=1,
                        device_id=(right,),
                        device_id_type=pl.DeviceIdType.MESH,
                    )

        buf_r[0, :, :] = partial(ring_pos(-1), 0).astype(jnp.bfloat16)
        buf_l[0, :, :] = partial(ring_pos(1), half).astype(jnp.bfloat16)
        for s in range(NSEG):
            mk(buf_r, send_r, recv_r, 0, s, right).start()
            mk(buf_l, send_l, recv_l, 0, s, left).start()

        hop(0, 0, mid=False, is_last=False)

        def four_hops(t, carry):
            h = 4 * t + 1
            hop(h, 1, mid=True, is_last=False)
            hop(h + 1, 2, mid=True, is_last=False)
            hop(h + 2, 3, mid=True, is_last=False)
            hop(h + 3, 0, mid=True, is_last=False)
            return carry

        lax.fori_loop(0, (N_DEV - 4) // NBUF, four_hops, 0)

        hop(N_DEV - 3, (N_DEV - 3) % NBUF, mid=True, is_last=False,
            credit=False)
        hop(N_DEV - 2, (N_DEV - 2) % NBUF, mid=False, is_last=True,
            credit=False)

        for s in range(NSEG):
            mk(buf_r, send_r, recv_r, (N_DEV - 3) % NBUF, s, right).wait_send()
            mk(buf_l, send_l, recv_l, (N_DEV - 3) % NBUF, s, left).wait_send()
            mk(buf_r, send_r, recv_r, (N_DEV - 2) % NBUF, s, right).wait_send()
            mk(buf_l, send_l, recv_l, (N_DEV - 2) % NBUF, s, left).wait_send()

    return pl.pallas_call(
        body,
        out_shape=jax.ShapeDtypeStruct((m_per, n), jnp.float32),
        in_specs=[
            pl.BlockSpec(memory_space=pltpu.SMEM),
            pl.BlockSpec(memory_space=pltpu.SMEM),
            pl.BlockSpec(memory_space=pltpu.VMEM),
            pl.BlockSpec(memory_space=pltpu.VMEM),
        ],
        out_specs=pl.BlockSpec(memory_space=pltpu.VMEM),
        scratch_shapes=[
            pltpu.VMEM((NBUF, m_per, half), jnp.bfloat16),
            pltpu.VMEM((NBUF, m_per, half), jnp.bfloat16),
            pltpu.SemaphoreType.DMA((NSEG, 2)),
            pltpu.SemaphoreType.DMA((NSEG, 2)),
            pltpu.SemaphoreType.DMA((NSEG, 2)),
            pltpu.SemaphoreType.DMA((NSEG, 2)),
            pltpu.SemaphoreType.REGULAR((NSEG, 2)),
            pltpu.SemaphoreType.REGULAR((NSEG, 2)),
        ],
        compiler_params=pltpu.CompilerParams(collective_id=0),
    )(r2p, p2r, x, w_mat)
